# baseline (speedup 1.0000x reference)
import functools

import jax
import jax.numpy as jnp
import numpy as np
from jax import lax
from jax.experimental import pallas as pl
from jax.experimental.pallas import tpu as pltpu
from jax.experimental.pallas import tpu_sc as plsc

_B, _L, _D = 1024, 200, 32
_SAMPLE_NUM = 10
_LM1 = _L - 1
_NNEG = _LM1 * _SAMPLE_NUM

_NC = 2
_NS = 16
_NW = _NC * _NS


def _sample_idx_table(L, sample_num, seed=0):
    rng = np.random.RandomState(seed)
    all_idx = [
        rng.choice([j for j in range(L) if j != idx_], size=sample_num, replace=False)
        for idx_ in range(L)
    ]
    return np.stack(all_idx[1:], axis=0).astype(np.int32)


def _routing_tables():
    flat = _sample_idx_table(_L, _SAMPLE_NUM).reshape(-1)
    counts = np.bincount(flat, minlength=_L).astype(np.int32)
    assert int(counts.max()) <= 31
    dtbl = np.zeros((_L, 32), np.int32)
    dtbl[:, 0] = counts
    fill = np.ones((_L,), np.int32)
    for k, j in enumerate(flat):
        dtbl[j, fill[j]] = k
        fill[j] += 1

    weight = counts + (np.arange(_L) >= 1)
    order = np.argsort(-weight, kind="stable")
    loads = np.zeros(_NW, np.int64)
    assign = [[] for _ in range(_NW)]
    for j in order:
        w = int(np.argmin(loads))
        loads[w] += int(weight[j])
        assign[w].append(int(j))
    max_slabs = max(len(a) for a in assign)
    assert max_slabs <= 15
    wtbl = np.zeros((_NW, 16), np.int32)
    for w, a in enumerate(assign):
        wtbl[w, 0] = len(a)
        wtbl[w, 1 : 1 + len(a)] = a
    return dtbl, wtbl


_DTBL, _WTBL = _routing_tables()

_mesh = plsc.VectorSubcoreMesh(core_axis_name="c", subcore_axis_name="s")


@functools.partial(
    pl.kernel,
    mesh=_mesh,
    out_type=(
        jax.ShapeDtypeStruct((_LM1, _D, _B), jnp.float32),
        jax.ShapeDtypeStruct((_NNEG, _D, _B), jnp.float32),
    ),
    scratch_types=[
        pltpu.VMEM((_L, 32), jnp.int32),
        pltpu.VMEM((_NW, 16), jnp.int32),
        pltpu.VMEM((2, _D, _B), jnp.float32),
        pltpu.SemaphoreType.DMA,
        pltpu.SemaphoreType.DMA,
    ],
    compiler_params=pltpu.CompilerParams(needs_layout_passes=False),
)
def _sc_route(
    x_hbm, dtbl_hbm, wtbl_hbm, pos_hbm, neg_hbm,
    dtbl_v, wtbl_v, slab_v, lsem, wsem,
):
    w = lax.axis_index("s") * _NC + lax.axis_index("c")
    pltpu.sync_copy(dtbl_hbm, dtbl_v)
    pltpu.sync_copy(wtbl_hbm, wtbl_v)
    lanes = lax.broadcasted_iota(jnp.int32, (16,), 0)
    wv = wtbl_v[w, pl.ds(0, 16)]

    def lane_of(vec, lane):
        return jnp.sum(jnp.where(lanes == lane, vec, 0))

    nslab = lane_of(wv, 0)
    j0 = lane_of(wv, 1)
    pltpu.async_copy(x_hbm.at[j0], slab_v.at[0], lsem)

    def slab_body(si, carry):
        p = si % 2
        j = lane_of(wv, si + 1)
        rv1 = dtbl_v[j, pl.ds(0, 16)]
        rv2 = dtbl_v[j, pl.ds(16, 16)]
        cnt = lane_of(rv1, 0)

        @pl.when(p == 0)
        def _():
            pltpu.make_async_copy(x_hbm.at[j], slab_v.at[0], lsem).wait()

        @pl.when(p == 1)
        def _():
            pltpu.make_async_copy(x_hbm.at[j], slab_v.at[1], lsem).wait()

        @pl.when(si + 1 < nslab)
        def _():
            jn = lane_of(wv, si + 2)

            @pl.when(p == 0)
            def _():
                pltpu.async_copy(x_hbm.at[jn], slab_v.at[1], lsem)

            @pl.when(p == 1)
            def _():
                pltpu.async_copy(x_hbm.at[jn], slab_v.at[0], lsem)

        def dest_body(c, carry2):
            cc = c + 1
            rv = jnp.where(cc < 16, rv1, rv2)
            lane = jnp.where(cc < 16, cc, cc - 16)
            d = jnp.sum(jnp.where(lanes == lane, rv, 0))

            @pl.when(p == 0)
            def _():
                pltpu.async_copy(slab_v.at[0], neg_hbm.at[d], wsem)

            @pl.when(p == 1)
            def _():
                pltpu.async_copy(slab_v.at[1], neg_hbm.at[d], wsem)

            return carry2

        lax.fori_loop(0, cnt, dest_body, 0)

        @pl.when((j >= 1) & (p == 0))
        def _():
            pltpu.async_copy(slab_v.at[0], pos_hbm.at[j - 1], wsem)

        @pl.when((j >= 1) & (p == 1))
        def _():
            pltpu.async_copy(slab_v.at[1], pos_hbm.at[j - 1], wsem)

        writes = cnt + jnp.where(j >= 1, 1, 0)

        def one(_, c):
            pltpu.make_async_copy(slab_v.at[0], neg_hbm.at[0], wsem).wait()
            return c

        lax.fori_loop(0, writes, one, 0)
        return carry

    lax.fori_loop(0, nslab, slab_body, 0)


def kernel(inputs):
    b, l, d = inputs.shape
    x_t = jnp.transpose(inputs, (1, 2, 0))
    pos_t, neg_t = _sc_route(x_t, jnp.asarray(_DTBL), jnp.asarray(_WTBL))
    pos = jnp.transpose(pos_t, (2, 0, 1))
    neg = jnp.transpose(
        neg_t.reshape(_LM1, _SAMPLE_NUM, d, b), (3, 0, 1, 2)
    )
    return pos, neg
